# Initial kernel scaffold; baseline (speedup 1.0000x reference)
import jax
import jax.numpy as jnp
from jax.experimental import pallas as pl


def kernel(x, y_table, slope):
    raise NotImplementedError("write your pallas kernel here")


# speedup vs baseline: 388.0527x; 388.0527x over previous
import functools

import jax
import jax.numpy as jnp
from jax import lax
from jax.experimental import pallas as pl
from jax.experimental.pallas import tpu as pltpu
from jax.experimental.pallas import tpu_sc as plsc

X_MIN = -100.0
X_MAX = 100.0
N_TAB = 50000
INV_STEP = 1.0 / ((X_MAX - X_MIN) / (N_TAB - 1))

NUM_WORKERS = 32
CHUNK = 4096
LANES = 16


def _compute_chunk(x_ref, o_ref, yt_ref, sl_ref):
    def vbody(i, carry):
        off = i * LANES
        xv = x_ref[pl.ds(off, LANES)]
        xc = jnp.minimum(jnp.maximum(xv, X_MIN), X_MAX)
        idx_f = (xc - X_MIN) * INV_STEP
        idx = idx_f.astype(jnp.int32)
        idx = jnp.minimum(jnp.maximum(idx, 0), N_TAB - 1)
        frac = idx_f - idx.astype(jnp.float32)
        yv = plsc.load_gather(yt_ref, [idx])
        mv = plsc.load_gather(sl_ref, [idx])
        res = yv + frac * mv
        res = jnp.where(xv > X_MAX, xv, res)
        res = jnp.where(xv < X_MIN, 0.0, res)
        o_ref[pl.ds(off, LANES)] = res
        return carry

    lax.fori_loop(0, CHUNK // LANES, vbody, 0, unroll=4)


def _gelu_sc(x_flat, y_table, slope):
    total = x_flat.shape[0]
    per_w = total // NUM_WORKERS
    n_chunks = per_w // CHUNK
    n_pairs = n_chunks // 2
    mesh = plsc.VectorSubcoreMesh(core_axis_name="c", subcore_axis_name="s")

    @functools.partial(
        pl.kernel,
        mesh=mesh,
        compiler_params=pltpu.CompilerParams(needs_layout_passes=False),
        out_type=jax.ShapeDtypeStruct((total,), jnp.float32),
        scratch_types=[
            pltpu.VMEM((N_TAB,), jnp.float32),
            pltpu.VMEM((N_TAB,), jnp.float32),
            pltpu.VMEM((CHUNK,), jnp.float32),
            pltpu.VMEM((CHUNK,), jnp.float32),
            pltpu.VMEM((CHUNK,), jnp.float32),
            pltpu.VMEM((CHUNK,), jnp.float32),
            pltpu.SemaphoreType.DMA,
            pltpu.SemaphoreType.DMA,
            pltpu.SemaphoreType.DMA,
            pltpu.SemaphoreType.DMA,
            pltpu.SemaphoreType.DMA,
        ],
    )
    def k(x_hbm, yt_hbm, sl_hbm, out_hbm, yt_v, sl_v, xin0_v, xin1_v,
          out0_v, out1_v, sem_in0, sem_in1, sem_out0, sem_out1, sem_tab):
        wid = lax.axis_index("s") * 2 + lax.axis_index("c")
        base = wid * per_w
        xin = (xin0_v, xin1_v)
        outb = (out0_v, out1_v)
        sems_in = (sem_in0, sem_in1)
        sems_out = (sem_out0, sem_out1)

        t1 = pltpu.async_copy(yt_hbm, yt_v, sem_tab)
        t2 = pltpu.async_copy(sl_hbm, sl_v, sem_tab)
        pltpu.async_copy(x_hbm.at[pl.ds(base, CHUNK)], xin[0], sems_in[0])
        pltpu.async_copy(x_hbm.at[pl.ds(base + CHUNK, CHUNK)], xin[1],
                         sems_in[1])
        t1.wait()
        t2.wait()

        def wait_in(b):
            pltpu.make_async_copy(x_hbm.at[pl.ds(0, CHUNK)], xin[b],
                                  sems_in[b]).wait()

        def wait_out(b):
            pltpu.make_async_copy(outb[b], out_hbm.at[pl.ds(0, CHUNK)],
                                  sems_out[b]).wait()

        def pair_body(p, carry):
            for b in range(2):
                g = 2 * p + b
                wait_in(b)

                @pl.when(p >= 1)
                def _():
                    wait_out(b)

                _compute_chunk(xin[b], outb[b], yt_v, sl_v)
                pltpu.async_copy(outb[b],
                                 out_hbm.at[pl.ds(base + g * CHUNK, CHUNK)],
                                 sems_out[b])

                @pl.when(p < n_pairs - 1)
                def _():
                    pltpu.async_copy(
                        x_hbm.at[pl.ds(base + (g + 2) * CHUNK, CHUNK)],
                        xin[b], sems_in[b])
            return carry

        lax.fori_loop(0, n_pairs, pair_body, 0)
        wait_out(0)
        wait_out(1)

    return k(x_flat, y_table, slope)


def kernel(x, y_table, slope):
    out = _gelu_sc(x.reshape(-1), y_table, slope)
    return out.reshape(x.shape)


# speedup vs baseline: 1175.2612x; 3.0286x over previous
import functools

import jax
import jax.numpy as jnp
from jax import lax
from jax.experimental import pallas as pl
from jax.experimental.pallas import tpu as pltpu
from jax.experimental.pallas import tpu_sc as plsc

X_MIN = -100.0
X_MAX = 100.0
N_TAB = 50000
INV_STEP = 1.0 / ((X_MAX - X_MIN) / (N_TAB - 1))

NUM_WORKERS = 32
CHUNK = 4096
LANES = 16


def _compute_chunk(x_ref, o_ref, yt_ref, sl_ref):
    @plsc.parallel_loop(0, CHUNK, step=LANES, unroll=8)
    def vbody(off):
        xv = x_ref[pl.ds(off, LANES)]
        xc = jnp.minimum(jnp.maximum(xv, X_MIN), X_MAX)
        idx_f = (xc - X_MIN) * INV_STEP
        idx = idx_f.astype(jnp.int32)
        idx = jnp.minimum(jnp.maximum(idx, 0), N_TAB - 1)
        frac = idx_f - idx.astype(jnp.float32)
        yv = plsc.load_gather(yt_ref, [idx])
        mv = plsc.load_gather(sl_ref, [idx])
        res = yv + frac * mv
        res = jnp.where(xv > X_MAX, xv, res)
        res = jnp.where(xv < X_MIN, 0.0, res)
        o_ref[pl.ds(off, LANES)] = res


def _gelu_sc(x_flat, y_table, slope):
    total = x_flat.shape[0]
    per_w = total // NUM_WORKERS
    n_chunks = per_w // CHUNK
    n_pairs = n_chunks // 2
    mesh = plsc.VectorSubcoreMesh(core_axis_name="c", subcore_axis_name="s")

    @functools.partial(
        pl.kernel,
        mesh=mesh,
        compiler_params=pltpu.CompilerParams(needs_layout_passes=False),
        out_type=jax.ShapeDtypeStruct((total,), jnp.float32),
        scratch_types=[
            pltpu.VMEM((N_TAB,), jnp.float32),
            pltpu.VMEM((N_TAB,), jnp.float32),
            pltpu.VMEM((CHUNK,), jnp.float32),
            pltpu.VMEM((CHUNK,), jnp.float32),
            pltpu.VMEM((CHUNK,), jnp.float32),
            pltpu.VMEM((CHUNK,), jnp.float32),
            pltpu.SemaphoreType.DMA,
            pltpu.SemaphoreType.DMA,
            pltpu.SemaphoreType.DMA,
            pltpu.SemaphoreType.DMA,
            pltpu.SemaphoreType.DMA,
        ],
    )
    def k(x_hbm, yt_hbm, sl_hbm, out_hbm, yt_v, sl_v, xin0_v, xin1_v,
          out0_v, out1_v, sem_in0, sem_in1, sem_out0, sem_out1, sem_tab):
        wid = lax.axis_index("s") * 2 + lax.axis_index("c")
        base = wid * per_w
        xin = (xin0_v, xin1_v)
        outb = (out0_v, out1_v)
        sems_in = (sem_in0, sem_in1)
        sems_out = (sem_out0, sem_out1)

        t1 = pltpu.async_copy(yt_hbm, yt_v, sem_tab)
        t2 = pltpu.async_copy(sl_hbm, sl_v, sem_tab)
        pltpu.async_copy(x_hbm.at[pl.ds(base, CHUNK)], xin[0], sems_in[0])
        pltpu.async_copy(x_hbm.at[pl.ds(base + CHUNK, CHUNK)], xin[1],
                         sems_in[1])
        t1.wait()
        t2.wait()

        def wait_in(b):
            pltpu.make_async_copy(x_hbm.at[pl.ds(0, CHUNK)], xin[b],
                                  sems_in[b]).wait()

        def wait_out(b):
            pltpu.make_async_copy(outb[b], out_hbm.at[pl.ds(0, CHUNK)],
                                  sems_out[b]).wait()

        def pair_body(p, carry):
            for b in range(2):
                g = 2 * p + b
                wait_in(b)

                @pl.when(p >= 1)
                def _():
                    wait_out(b)

                _compute_chunk(xin[b], outb[b], yt_v, sl_v)
                pltpu.async_copy(outb[b],
                                 out_hbm.at[pl.ds(base + g * CHUNK, CHUNK)],
                                 sems_out[b])

                @pl.when(p < n_pairs - 1)
                def _():
                    pltpu.async_copy(
                        x_hbm.at[pl.ds(base + (g + 2) * CHUNK, CHUNK)],
                        xin[b], sems_in[b])
            return carry

        lax.fori_loop(0, n_pairs, pair_body, 0)
        wait_out(0)
        wait_out(1)

    return k(x_flat, y_table, slope)


def kernel(x, y_table, slope):
    out = _gelu_sc(x.reshape(-1), y_table, slope)
    return out.reshape(x.shape)


# speedup vs baseline: 1300.0775x; 1.1062x over previous
import functools

import jax
import jax.numpy as jnp
from jax import lax
from jax.experimental import pallas as pl
from jax.experimental.pallas import tpu as pltpu
from jax.experimental.pallas import tpu_sc as plsc

X_MIN = -100.0
X_MAX = 100.0
N_TAB = 50000
INV_STEP = 1.0 / ((X_MAX - X_MIN) / (N_TAB - 1))

NUM_WORKERS = 32
CHUNK = 4096
LANES = 16


def _compute_chunk(x_ref, o_ref, yt_ref, sl_ref):
    @plsc.parallel_loop(0, CHUNK, step=LANES, unroll=8)
    def vbody(off):
        xv = x_ref[pl.ds(off, LANES)]
        idx_f = (xv - X_MIN) * INV_STEP
        idx_f = jnp.minimum(jnp.maximum(idx_f, 0.0), float(N_TAB - 1))
        idx = idx_f.astype(jnp.int32)
        frac = idx_f - idx.astype(jnp.float32)
        yv = plsc.load_gather(yt_ref, [idx])
        mv = plsc.load_gather(sl_ref, [idx])
        res = yv + frac * mv
        res = jnp.where(xv > X_MAX, xv, res)
        o_ref[pl.ds(off, LANES)] = res


def _gelu_sc(x_flat, y_table, slope):
    total = x_flat.shape[0]
    per_w = total // NUM_WORKERS
    n_chunks = per_w // CHUNK
    n_pairs = n_chunks // 2
    mesh = plsc.VectorSubcoreMesh(core_axis_name="c", subcore_axis_name="s")

    @functools.partial(
        pl.kernel,
        mesh=mesh,
        compiler_params=pltpu.CompilerParams(needs_layout_passes=False),
        out_type=jax.ShapeDtypeStruct((total,), jnp.float32),
        scratch_types=[
            pltpu.VMEM((N_TAB,), jnp.float32),
            pltpu.VMEM((N_TAB,), jnp.float32),
            pltpu.VMEM((CHUNK,), jnp.float32),
            pltpu.VMEM((CHUNK,), jnp.float32),
            pltpu.VMEM((CHUNK,), jnp.float32),
            pltpu.VMEM((CHUNK,), jnp.float32),
            pltpu.SemaphoreType.DMA,
            pltpu.SemaphoreType.DMA,
            pltpu.SemaphoreType.DMA,
            pltpu.SemaphoreType.DMA,
            pltpu.SemaphoreType.DMA,
        ],
    )
    def k(x_hbm, yt_hbm, sl_hbm, out_hbm, yt_v, sl_v, xin0_v, xin1_v,
          out0_v, out1_v, sem_in0, sem_in1, sem_out0, sem_out1, sem_tab):
        wid = lax.axis_index("s") * 2 + lax.axis_index("c")
        base = wid * per_w
        xin = (xin0_v, xin1_v)
        outb = (out0_v, out1_v)
        sems_in = (sem_in0, sem_in1)
        sems_out = (sem_out0, sem_out1)

        t1 = pltpu.async_copy(yt_hbm, yt_v, sem_tab)
        t2 = pltpu.async_copy(sl_hbm, sl_v, sem_tab)
        pltpu.async_copy(x_hbm.at[pl.ds(base, CHUNK)], xin[0], sems_in[0])
        pltpu.async_copy(x_hbm.at[pl.ds(base + CHUNK, CHUNK)], xin[1],
                         sems_in[1])
        t1.wait()
        t2.wait()

        def wait_in(b):
            pltpu.make_async_copy(x_hbm.at[pl.ds(0, CHUNK)], xin[b],
                                  sems_in[b]).wait()

        def wait_out(b):
            pltpu.make_async_copy(outb[b], out_hbm.at[pl.ds(0, CHUNK)],
                                  sems_out[b]).wait()

        def pair_body(p, carry):
            for b in range(2):
                g = 2 * p + b
                wait_in(b)

                @pl.when(p >= 1)
                def _():
                    wait_out(b)

                _compute_chunk(xin[b], outb[b], yt_v, sl_v)
                pltpu.async_copy(outb[b],
                                 out_hbm.at[pl.ds(base + g * CHUNK, CHUNK)],
                                 sems_out[b])

                @pl.when(p < n_pairs - 1)
                def _():
                    pltpu.async_copy(
                        x_hbm.at[pl.ds(base + (g + 2) * CHUNK, CHUNK)],
                        xin[b], sems_in[b])
            return carry

        lax.fori_loop(0, n_pairs, pair_body, 0)
        wait_out(0)
        wait_out(1)

    return k(x_flat, y_table, slope)


def kernel(x, y_table, slope):
    out = _gelu_sc(x.reshape(-1), y_table, slope)
    return out.reshape(x.shape)


# speedup vs baseline: 2456.8850x; 1.8898x over previous
import functools

import jax
import jax.numpy as jnp
from jax import lax
from jax.experimental import pallas as pl
from jax.experimental.pallas import tpu as pltpu
from jax.experimental.pallas import tpu_sc as plsc

X_MIN = -100.0
X_MAX = 100.0
N_TAB = 50000
INV_STEP = 1.0 / ((X_MAX - X_MIN) / (N_TAB - 1))

NUM_WORKERS = 32
LANES = 16
BLK_R = 8
BLK_C = 512
CHUNK = BLK_R * BLK_C


def _compute_chunk(x_ref, o_ref, yt_ref, sl_ref):
    @plsc.parallel_loop(0, CHUNK, step=LANES, unroll=8)
    def vbody(off):
        r = off >> 9
        c = off & (BLK_C - 1)
        xv = x_ref[r, pl.ds(c, LANES)]
        idx_f = (xv - X_MIN) * INV_STEP
        idx_f = jnp.minimum(jnp.maximum(idx_f, 0.0), float(N_TAB - 1))
        idx = idx_f.astype(jnp.int32)
        frac = idx_f - idx.astype(jnp.float32)
        yv = plsc.load_gather(yt_ref, [idx])
        mv = plsc.load_gather(sl_ref, [idx])
        res = yv + frac * mv
        res = jnp.where(xv > X_MAX, xv, res)
        o_ref[r, pl.ds(c, LANES)] = res


def _gelu_sc(x, y_table, slope):
    slab, rows, cols = x.shape
    band = rows * slab // NUM_WORKERS
    row_blocks = band // BLK_R
    col_blocks = cols // BLK_C
    n_chunks = row_blocks * col_blocks
    n_pairs = n_chunks // 2
    mesh = plsc.VectorSubcoreMesh(core_axis_name="c", subcore_axis_name="s")

    @functools.partial(
        pl.kernel,
        mesh=mesh,
        compiler_params=pltpu.CompilerParams(needs_layout_passes=False,
                                             use_tc_tiling_on_sc=True),
        out_type=jax.ShapeDtypeStruct((slab, rows, cols), jnp.float32),
        scratch_types=[
            pltpu.VMEM((N_TAB,), jnp.float32),
            pltpu.VMEM((N_TAB,), jnp.float32),
            pltpu.VMEM((BLK_R, BLK_C), jnp.float32),
            pltpu.VMEM((BLK_R, BLK_C), jnp.float32),
            pltpu.VMEM((BLK_R, BLK_C), jnp.float32),
            pltpu.VMEM((BLK_R, BLK_C), jnp.float32),
            pltpu.SemaphoreType.DMA,
            pltpu.SemaphoreType.DMA,
            pltpu.SemaphoreType.DMA,
            pltpu.SemaphoreType.DMA,
            pltpu.SemaphoreType.DMA,
        ],
    )
    def k(x_hbm, yt_hbm, sl_hbm, out_hbm, yt_v, sl_v, xin0_v, xin1_v,
          out0_v, out1_v, sem_in0, sem_in1, sem_out0, sem_out1, sem_tab):
        wid = lax.axis_index("s") * 2 + lax.axis_index("c")
        d = wid // (NUM_WORKERS // slab)
        row0 = (wid % (NUM_WORKERS // slab)) * band
        xin = (xin0_v, xin1_v)
        outb = (out0_v, out1_v)
        sems_in = (sem_in0, sem_in1)
        sems_out = (sem_out0, sem_out1)

        def src(g):
            a = g // col_blocks
            b_ = g % col_blocks
            return (d, pl.ds(row0 + BLK_R * a, BLK_R),
                    pl.ds(BLK_C * b_, BLK_C))

        t1 = pltpu.async_copy(yt_hbm, yt_v, sem_tab)
        t2 = pltpu.async_copy(sl_hbm, sl_v, sem_tab)
        pltpu.async_copy(x_hbm.at[src(0)], xin[0], sems_in[0])
        pltpu.async_copy(x_hbm.at[src(1)], xin[1], sems_in[1])
        t1.wait()
        t2.wait()

        def wait_in(b):
            pltpu.make_async_copy(x_hbm.at[src(0)], xin[b],
                                  sems_in[b]).wait()

        def wait_out(b):
            pltpu.make_async_copy(outb[b], out_hbm.at[src(0)],
                                  sems_out[b]).wait()

        def pair_body(p, carry):
            for b in range(2):
                g = 2 * p + b
                wait_in(b)

                @pl.when(p >= 1)
                def _():
                    wait_out(b)

                _compute_chunk(xin[b], outb[b], yt_v, sl_v)
                pltpu.async_copy(outb[b], out_hbm.at[src(g)], sems_out[b])

                @pl.when(p < n_pairs - 1)
                def _():
                    pltpu.async_copy(x_hbm.at[src(g + 2)], xin[b],
                                     sems_in[b])
            return carry

        lax.fori_loop(0, n_pairs, pair_body, 0)
        wait_out(0)
        wait_out(1)

    return k(x, y_table, slope)


def kernel(x, y_table, slope):
    return _gelu_sc(x, y_table, slope)
